# baseline (speedup 1.0000x reference)
import functools

import jax
import jax.numpy as jnp
from jax import lax
from jax.experimental import pallas as pl
from jax.experimental.pallas import tpu as pltpu
from jax.experimental.pallas import tpu_sc as plsc

NUM_SAMPLE = 4096
S = 8192
NW = 32
R = NUM_SAMPLE // NW
C0 = 128
CC = 128
LANES = 16


def _prelude_body(ifb_ref, iast_ref, exp_u_ref, tle_ref, bnd_ref, ratio_ref,
                  times_ref, tot_ref, sc_ref, w_ref):
    r = ratio_ref[0, 0]
    tle = tle_ref[0, 0]
    bnd = bnd_ref[0, 0]
    bounds = jnp.max(jnp.sum(ifb_ref[...], axis=-1)) * 5.0
    sr = bounds * r
    expn = -jnp.log1p(-jnp.clip(exp_u_ref[...], 0.0, 1.0 - 1e-7))
    x = expn / sr
    lane = lax.broadcasted_iota(jnp.int32, (1, S), 1)
    sh = 1
    while sh < S:
        rolled = pltpu.roll(x, sh, axis=1)
        x = x + jnp.where(lane >= sh, rolled, 0.0)
        sh *= 2
    times = x + tle
    times_ref[...] = times
    tot_ref[...] = jnp.sum(iast_ref[...], axis=1, keepdims=True) * r
    last = jnp.max(times)
    big = last + 1.0
    fb = jnp.where(last > bnd, last, bnd)
    li = lax.broadcasted_iota(jnp.int32, (1, 48), 1)
    sc_ref[...] = jnp.where(li < 16, sr, jnp.where(li < 32, big, fb))
    w_ref[...] = jnp.full((1, NUM_SAMPLE), 1.0 / NUM_SAMPLE, jnp.float32)


def _prelude(ifb, iast2, exp_u, tle, bnd, ratio):
    return pl.pallas_call(
        _prelude_body,
        in_specs=[
            pl.BlockSpec(memory_space=pltpu.VMEM),
            pl.BlockSpec(memory_space=pltpu.VMEM),
            pl.BlockSpec(memory_space=pltpu.VMEM),
            pl.BlockSpec(memory_space=pltpu.SMEM),
            pl.BlockSpec(memory_space=pltpu.SMEM),
            pl.BlockSpec(memory_space=pltpu.SMEM),
        ],
        out_specs=[
            pl.BlockSpec(memory_space=pltpu.VMEM),
            pl.BlockSpec(memory_space=pltpu.VMEM),
            pl.BlockSpec(memory_space=pltpu.VMEM),
            pl.BlockSpec(memory_space=pltpu.VMEM),
        ],
        out_shape=[
            jax.ShapeDtypeStruct((1, S), jnp.float32),
            jax.ShapeDtypeStruct((S, 1), jnp.float32),
            jax.ShapeDtypeStruct((1, 48), jnp.float32),
            jax.ShapeDtypeStruct((1, NUM_SAMPLE), jnp.float32),
        ],
    )(ifb, iast2, exp_u, tle, bnd, ratio)


@functools.partial(
    pl.kernel,
    out_type=jax.ShapeDtypeStruct((NUM_SAMPLE,), jnp.float32),
    mesh=plsc.VectorSubcoreMesh(core_axis_name="c", subcore_axis_name="s"),
    scratch_types=[
        pltpu.VMEM((48,), jnp.float32),
        pltpu.VMEM((S,), jnp.float32),
        pltpu.VMEM((S,), jnp.float32),
        pltpu.VMEM((R, C0), jnp.float32),
        pltpu.VMEM((CC,), jnp.float32),
        pltpu.VMEM((R,), jnp.float32),
        pltpu.SMEM((R,), jnp.int32),
        pltpu.SMEM((1,), jnp.int32),
        pltpu.SemaphoreType.DMA,
    ],
)
def _scan_kernel(sc_hbm, tot_hbm, times_hbm, unif_hbm, unif_flat_hbm, rst_hbm,
                 sc_v, tot_v, times_v, buf_v, cbuf_v, rst_v, list_s, cnt_s,
                 sem):
    wid = lax.axis_index("s") * 2 + lax.axis_index("c")
    base = wid * R
    cp = pltpu.async_copy(unif_hbm.at[pl.ds(base, R), pl.ds(0, C0)], buf_v, sem)
    pltpu.sync_copy(sc_hbm, sc_v)
    pltpu.sync_copy(tot_hbm, tot_v)
    pltpu.sync_copy(times_hbm, times_v)
    srv = sc_v[pl.ds(0, LANES)]
    bigv = sc_v[pl.ds(16, LANES)]
    fbv = sc_v[pl.ds(32, LANES)]
    lane = lax.broadcasted_iota(jnp.int32, (LANES,), 0)
    lane0 = lane == 0
    cnt_s[0] = 0
    cp.wait()

    def lane_min(x):
        for k in (1, 2, 4, 8):
            x = jnp.minimum(x, x.at[lane ^ k].get(mode="promise_in_bounds"))
        return x

    def fold(load_u, col0):
        acc = bigv
        for v in range(CC // LANES):
            u = load_u(v)
            t = tot_v[pl.ds(col0 + v * LANES, LANES)]
            tm = times_v[pl.ds(col0 + v * LANES, LANES)]
            crit = u * srv / t
            acc = jnp.minimum(acc, jnp.where(crit < 1.0, tm, bigv))
        return acc

    def scalar_of(vec):
        return jnp.reshape(lax.slice(vec, (0,), (1,)), ())

    def append_if_unresolved(row, minv):
        c = cnt_s[0]
        list_s[c] = row
        cnt_s[0] = c + scalar_of(jnp.where(minv < bigv, 0, 1))

    def result_of(minv):
        return jnp.where(minv < bigv, minv, fbv)

    def row_fn(i, gvec):
        acc = fold(lambda v: buf_v[i, pl.ds(v * LANES, LANES)], 0)
        minv = lane_min(acc)
        append_if_unresolved(i, minv)
        gvec = jnp.where(lane == lax.rem(i, LANES), result_of(minv), gvec)

        @pl.when(lax.rem(i, LANES) == LANES - 1)
        def _():
            rst_v[pl.ds((i // LANES) * LANES, LANES)] = gvec

        return gvec

    lax.fori_loop(0, R, row_fn, jnp.zeros((LANES,), jnp.float32))

    def round_fn(r, carry):
        n = cnt_s[0]
        cnt_s[0] = 0
        col = C0 + r * CC

        def item_fn(k, carry2):
            row = list_s[k]
            off = pl.multiple_of((base + row) * S + col, 8)
            pltpu.sync_copy(unif_flat_hbm.at[pl.ds(off, CC)], cbuf_v)
            acc = fold(lambda v: cbuf_v[pl.ds(v * LANES, LANES)], col)
            minv = lane_min(acc)
            append_if_unresolved(row, minv)
            g16 = (row // LANES) * LANES
            old = rst_v[pl.ds(g16, LANES)]
            sel = lane == lax.rem(row, LANES)
            rst_v[pl.ds(g16, LANES)] = jnp.where(sel, result_of(minv), old)
            return carry2

        lax.fori_loop(0, n, item_fn, 0)
        return carry

    lax.fori_loop(0, (S - C0) // CC, round_fn, 0)
    pltpu.sync_copy(rst_v, rst_hbm.at[pl.ds(base, R)])


def kernel(intensities_for_bound, intensities_at_sampled_times, exp_u,
           unif_numbers, time_last_event, boundary, ratio):
    iast2 = intensities_at_sampled_times.reshape(S, intensities_at_sampled_times.shape[-1])
    times, tot, sc, w = _prelude(
        intensities_for_bound, iast2, exp_u,
        time_last_event.reshape(1, 1), boundary.reshape(1, 1),
        ratio.reshape(1, 1))
    rst = _scan_kernel(sc.reshape(48), tot.reshape(S), times.reshape(S),
                       unif_numbers, unif_numbers.reshape(NUM_SAMPLE * S))
    return rst, w.reshape(NUM_SAMPLE)


# speedup vs baseline: 2.9657x; 2.9657x over previous
import functools

import jax
import jax.numpy as jnp
from jax import lax
from jax.experimental import pallas as pl
from jax.experimental.pallas import tpu as pltpu
from jax.experimental.pallas import tpu_sc as plsc

NUM_SAMPLE = 4096
S = 8192
NW = 32
R = NUM_SAMPLE // NW
C0 = 128
CC = 128
LANES = 16


def _prelude_body(ifb_ref, iast_ref, exp_u_ref, tle_ref, bnd_ref, ratio_ref,
                  times_ref, tot_ref, sc_ref, w_ref):
    r = ratio_ref[0, 0]
    tle = tle_ref[0, 0]
    bnd = bnd_ref[0, 0]
    bounds = jnp.max(jnp.sum(ifb_ref[...], axis=-1)) * 5.0
    sr = bounds * r
    expn = -jnp.log1p(-jnp.clip(exp_u_ref[...], 0.0, 1.0 - 1e-7))
    x = expn / sr
    lane = lax.broadcasted_iota(jnp.int32, (1, S), 1)
    sh = 1
    while sh < S:
        rolled = pltpu.roll(x, sh, axis=1)
        x = x + jnp.where(lane >= sh, rolled, 0.0)
        sh *= 2
    times = x + tle
    times_ref[...] = times
    tot_ref[...] = jnp.sum(iast_ref[...], axis=1, keepdims=True) * r
    last = jnp.max(times)
    big = last + 1.0
    fb = jnp.where(last > bnd, last, bnd)
    li = lax.broadcasted_iota(jnp.int32, (1, 48), 1)
    sc_ref[...] = jnp.where(li < 16, sr, jnp.where(li < 32, big, fb))
    w_ref[...] = jnp.full((1, NUM_SAMPLE), 1.0 / NUM_SAMPLE, jnp.float32)


def _prelude(ifb, iast2, exp_u, tle, bnd, ratio):
    return pl.pallas_call(
        _prelude_body,
        in_specs=[
            pl.BlockSpec(memory_space=pltpu.VMEM),
            pl.BlockSpec(memory_space=pltpu.VMEM),
            pl.BlockSpec(memory_space=pltpu.VMEM),
            pl.BlockSpec(memory_space=pltpu.SMEM),
            pl.BlockSpec(memory_space=pltpu.SMEM),
            pl.BlockSpec(memory_space=pltpu.SMEM),
        ],
        out_specs=[
            pl.BlockSpec(memory_space=pltpu.VMEM),
            pl.BlockSpec(memory_space=pltpu.VMEM),
            pl.BlockSpec(memory_space=pltpu.VMEM),
            pl.BlockSpec(memory_space=pltpu.VMEM),
        ],
        out_shape=[
            jax.ShapeDtypeStruct((1, S), jnp.float32),
            jax.ShapeDtypeStruct((S, 1), jnp.float32),
            jax.ShapeDtypeStruct((1, 48), jnp.float32),
            jax.ShapeDtypeStruct((1, NUM_SAMPLE), jnp.float32),
        ],
    )(ifb, iast2, exp_u, tle, bnd, ratio)


@functools.partial(
    pl.kernel,
    out_type=jax.ShapeDtypeStruct((NUM_SAMPLE,), jnp.float32),
    mesh=plsc.VectorSubcoreMesh(core_axis_name="c", subcore_axis_name="s"),
    scratch_types=[
        pltpu.VMEM((48,), jnp.float32),
        pltpu.VMEM((S,), jnp.float32),
        pltpu.VMEM((S,), jnp.float32),
        pltpu.VMEM((R, C0), jnp.float32),
        pltpu.VMEM((8, CC), jnp.float32),
        pltpu.VMEM((R,), jnp.float32),
        pltpu.SMEM((R,), jnp.int32),
        pltpu.SMEM((1,), jnp.int32),
        pltpu.SemaphoreType.DMA,
    ],
)
def _scan_kernel(sc_hbm, tot_hbm, times_hbm, unif_hbm, rst_hbm,
                 sc_v, tot_v, times_v, buf_v, cbuf_v, rst_v, list_s, cnt_s,
                 sem):
    wid = lax.axis_index("s") * 2 + lax.axis_index("c")
    base = wid * R
    cp = pltpu.async_copy(unif_hbm.at[pl.ds(base, R), pl.ds(0, C0)], buf_v, sem)
    pltpu.sync_copy(sc_hbm, sc_v)
    pltpu.sync_copy(tot_hbm, tot_v)
    pltpu.sync_copy(times_hbm, times_v)
    srv = sc_v[pl.ds(0, LANES)]
    bigv = sc_v[pl.ds(16, LANES)]
    fbv = sc_v[pl.ds(32, LANES)]
    lane = lax.broadcasted_iota(jnp.int32, (LANES,), 0)
    lane0 = lane == 0
    cnt_s[0] = 0
    cp.wait()

    def lane_min(x):
        for k in (1, 2, 4, 8):
            x = jnp.minimum(x, x.at[lane ^ k].get(mode="promise_in_bounds"))
        return x

    def fold(load_u, col0):
        acc = bigv
        for v in range(CC // LANES):
            u = load_u(v)
            t = tot_v[pl.ds(col0 + v * LANES, LANES)]
            tm = times_v[pl.ds(col0 + v * LANES, LANES)]
            crit = u * srv / t
            acc = jnp.minimum(acc, jnp.where(crit < 1.0, tm, bigv))
        return acc

    def scalar_of(vec):
        return jnp.reshape(lax.slice(vec, (0,), (1,)), ())

    def append_if_unresolved(row, minv):
        c = cnt_s[0]
        list_s[c] = row
        cnt_s[0] = c + scalar_of(jnp.where(minv < bigv, 0, 1))

    def result_of(minv):
        return jnp.where(minv < bigv, minv, fbv)

    def row_fn(i, gvec):
        acc = fold(lambda v: buf_v[i, pl.ds(v * LANES, LANES)], 0)
        minv = lane_min(acc)
        append_if_unresolved(i, minv)
        gvec = jnp.where(lane == lax.rem(i, LANES), result_of(minv), gvec)

        @pl.when(lax.rem(i, LANES) == LANES - 1)
        def _():
            rst_v[pl.ds((i // LANES) * LANES, LANES)] = gvec

        return gvec

    lax.fori_loop(0, R, row_fn, jnp.zeros((LANES,), jnp.float32))

    def round_fn(r, carry):
        n = cnt_s[0]
        cnt_s[0] = 0
        col = C0 + r * CC

        def item_fn(k, carry2):
            row = list_s[k]
            r8 = pl.multiple_of(base + (row // 8) * 8, 8)
            pltpu.sync_copy(unif_hbm.at[pl.ds(r8, 8), pl.ds(col, CC)], cbuf_v)
            sub = lax.rem(row, 8)
            acc = fold(lambda v: cbuf_v[sub, pl.ds(v * LANES, LANES)], col)
            minv = lane_min(acc)
            append_if_unresolved(row, minv)
            g16 = (row // LANES) * LANES
            old = rst_v[pl.ds(g16, LANES)]
            sel = lane == lax.rem(row, LANES)
            rst_v[pl.ds(g16, LANES)] = jnp.where(sel, result_of(minv), old)
            return carry2

        lax.fori_loop(0, n, item_fn, 0)
        return carry

    lax.fori_loop(0, (S - C0) // CC, round_fn, 0)
    pltpu.sync_copy(rst_v, rst_hbm.at[pl.ds(base, R)])


def kernel(intensities_for_bound, intensities_at_sampled_times, exp_u,
           unif_numbers, time_last_event, boundary, ratio):
    iast2 = intensities_at_sampled_times.reshape(S, intensities_at_sampled_times.shape[-1])
    times, tot, sc, w = _prelude(
        intensities_for_bound, iast2, exp_u,
        time_last_event.reshape(1, 1), boundary.reshape(1, 1),
        ratio.reshape(1, 1))
    rst = _scan_kernel(sc.reshape(48), tot.reshape(S), times.reshape(S),
                       unif_numbers)
    return rst, w.reshape(NUM_SAMPLE)


# speedup vs baseline: 3.9839x; 1.3433x over previous
import functools

import jax
import jax.numpy as jnp
from jax import lax
from jax.experimental import pallas as pl
from jax.experimental.pallas import tpu as pltpu
from jax.experimental.pallas import tpu_sc as plsc

NUM_SAMPLE = 4096
S = 8192
NW = 32
R = NUM_SAMPLE // NW
C0 = 128
CC = 128
LANES = 16


def _prelude_body(ifb_ref, iast_ref, exp_u_ref, tle_ref, bnd_ref, ratio_ref,
                  times_ref, tot_ref, sc_ref, w_ref):
    r = ratio_ref[0, 0]
    tle = tle_ref[0, 0]
    bnd = bnd_ref[0, 0]
    bounds = jnp.max(jnp.sum(ifb_ref[...], axis=-1)) * 5.0
    sr = bounds * r
    expn = -jnp.log1p(-jnp.clip(exp_u_ref[...], 0.0, 1.0 - 1e-7))
    x = expn / sr
    lane = lax.broadcasted_iota(jnp.int32, (1, S), 1)
    sh = 1
    while sh < S:
        rolled = pltpu.roll(x, sh, axis=1)
        x = x + jnp.where(lane >= sh, rolled, 0.0)
        sh *= 2
    times = x + tle
    times_ref[...] = jnp.reshape(times, (S,))
    tot = jnp.sum(iast_ref[...], axis=0, keepdims=True) * r
    tot_ref[...] = jnp.reshape(tot, (S,))
    last = jnp.max(times)
    big = last + 1.0
    fb = jnp.where(last > bnd, last, bnd)
    li = lax.broadcasted_iota(jnp.int32, (1, 48), 1)
    sc_ref[...] = jnp.reshape(
        jnp.where(li < 16, sr, jnp.where(li < 32, big, fb)), (48,))
    w_ref[...] = jnp.full((NUM_SAMPLE,), 1.0 / NUM_SAMPLE, jnp.float32)


def _prelude(ifb, iast_t, exp_u, tle, bnd, ratio):
    return pl.pallas_call(
        _prelude_body,
        in_specs=[
            pl.BlockSpec(),
            pl.BlockSpec(),
            pl.BlockSpec(),
            pl.BlockSpec(memory_space=pltpu.SMEM),
            pl.BlockSpec(memory_space=pltpu.SMEM),
            pl.BlockSpec(memory_space=pltpu.SMEM),
        ],
        out_shape=[
            jax.ShapeDtypeStruct((S,), jnp.float32),
            jax.ShapeDtypeStruct((S,), jnp.float32),
            jax.ShapeDtypeStruct((48,), jnp.float32),
            jax.ShapeDtypeStruct((NUM_SAMPLE,), jnp.float32),
        ],
    )(ifb, iast_t, exp_u, tle, bnd, ratio)


@functools.partial(
    pl.kernel,
    out_type=jax.ShapeDtypeStruct((NUM_SAMPLE,), jnp.float32),
    mesh=plsc.VectorSubcoreMesh(core_axis_name="c", subcore_axis_name="s"),
    scratch_types=[
        pltpu.VMEM((48,), jnp.float32),
        pltpu.VMEM((S,), jnp.float32),
        pltpu.VMEM((S,), jnp.float32),
        pltpu.VMEM((R, C0), jnp.float32),
        pltpu.VMEM((8, CC), jnp.float32),
        pltpu.VMEM((R,), jnp.float32),
        pltpu.SMEM((R,), jnp.int32),
        pltpu.SMEM((1,), jnp.int32),
        pltpu.SemaphoreType.DMA,
    ],
)
def _scan_kernel(sc_hbm, tot_hbm, times_hbm, unif_hbm, rst_hbm,
                 sc_v, tot_v, times_v, buf_v, cbuf_v, rst_v, list_s, cnt_s,
                 sem):
    wid = lax.axis_index("s") * 2 + lax.axis_index("c")
    base = wid * R
    cp = pltpu.async_copy(unif_hbm.at[pl.ds(base, R), pl.ds(0, C0)], buf_v, sem)
    pltpu.sync_copy(sc_hbm, sc_v)
    pltpu.sync_copy(tot_hbm, tot_v)
    pltpu.sync_copy(times_hbm, times_v)
    srv = sc_v[pl.ds(0, LANES)]
    bigv = sc_v[pl.ds(16, LANES)]
    fbv = sc_v[pl.ds(32, LANES)]
    lane = lax.broadcasted_iota(jnp.int32, (LANES,), 0)
    lane0 = lane == 0
    cnt_s[0] = 0
    cp.wait()

    def lane_min(x):
        for k in (1, 2, 4, 8):
            x = jnp.minimum(x, x.at[lane ^ k].get(mode="promise_in_bounds"))
        return x

    def fold(load_u, col0):
        acc = bigv
        for v in range(CC // LANES):
            u = load_u(v)
            t = tot_v[pl.ds(col0 + v * LANES, LANES)]
            tm = times_v[pl.ds(col0 + v * LANES, LANES)]
            crit = u * srv / t
            acc = jnp.minimum(acc, jnp.where(crit < 1.0, tm, bigv))
        return acc

    def scalar_of(vec):
        return jnp.reshape(lax.slice(vec, (0,), (1,)), ())

    def append_if_unresolved(row, minv):
        c = cnt_s[0]
        list_s[c] = row
        cnt_s[0] = c + scalar_of(jnp.where(minv < bigv, 0, 1))

    def result_of(minv):
        return jnp.where(minv < bigv, minv, fbv)

    def row_fn(i, gvec):
        acc = fold(lambda v: buf_v[i, pl.ds(v * LANES, LANES)], 0)
        minv = lane_min(acc)
        append_if_unresolved(i, minv)
        gvec = jnp.where(lane == lax.rem(i, LANES), result_of(minv), gvec)

        @pl.when(lax.rem(i, LANES) == LANES - 1)
        def _():
            rst_v[pl.ds((i // LANES) * LANES, LANES)] = gvec

        return gvec

    lax.fori_loop(0, R, row_fn, jnp.zeros((LANES,), jnp.float32))

    def round_fn(r, carry):
        n = cnt_s[0]
        cnt_s[0] = 0
        col = C0 + r * CC

        def item_fn(k, carry2):
            row = list_s[k]
            r8 = pl.multiple_of(base + (row // 8) * 8, 8)
            pltpu.sync_copy(unif_hbm.at[pl.ds(r8, 8), pl.ds(col, CC)], cbuf_v)
            sub = lax.rem(row, 8)
            acc = fold(lambda v: cbuf_v[sub, pl.ds(v * LANES, LANES)], col)
            minv = lane_min(acc)
            append_if_unresolved(row, minv)
            g16 = (row // LANES) * LANES
            old = rst_v[pl.ds(g16, LANES)]
            sel = lane == lax.rem(row, LANES)
            rst_v[pl.ds(g16, LANES)] = jnp.where(sel, result_of(minv), old)
            return carry2

        lax.fori_loop(0, n, item_fn, 0)
        return carry

    lax.fori_loop(0, (S - C0) // CC, round_fn, 0)
    pltpu.sync_copy(rst_v, rst_hbm.at[pl.ds(base, R)])


def kernel(intensities_for_bound, intensities_at_sampled_times, exp_u,
           unif_numbers, time_last_event, boundary, ratio):
    iast_t = intensities_at_sampled_times.reshape(S, -1).T
    times, tot, sc, w = _prelude(
        intensities_for_bound, iast_t, exp_u,
        time_last_event.reshape(1, 1), boundary.reshape(1, 1),
        ratio.reshape(1, 1))
    rst = _scan_kernel(sc, tot, times, unif_numbers)
    return rst, w


# speedup vs baseline: 4.0439x; 1.0150x over previous
import functools

import jax
import jax.numpy as jnp
from jax import lax
from jax.experimental import pallas as pl
from jax.experimental.pallas import tpu as pltpu
from jax.experimental.pallas import tpu_sc as plsc

NUM_SAMPLE = 4096
S = 8192
NW = 32
R = NUM_SAMPLE // NW
C0 = 128
CC = 128
LANES = 16


def _prelude_body(ifb_ref, iast_ref, exp_u_ref, tle_ref, bnd_ref, ratio_ref,
                  out_ref):
    r = ratio_ref[0, 0]
    tle = tle_ref[0, 0]
    bnd = bnd_ref[0, 0]
    bounds = jnp.max(jnp.sum(ifb_ref[...], axis=-1)) * 5.0
    sr = bounds * r
    expn = -jnp.log1p(-jnp.clip(exp_u_ref[...], 0.0, 1.0 - 1e-7))
    x = expn / sr
    lane = lax.broadcasted_iota(jnp.int32, (1, S), 1)
    sh = 1
    while sh < S:
        rolled = pltpu.roll(x, sh, axis=1)
        x = x + jnp.where(lane >= sh, rolled, 0.0)
        sh *= 2
    times = x + tle
    out_ref[pl.ds(0, S)] = jnp.reshape(times, (S,))
    thr = jnp.sum(iast_ref[...], axis=0, keepdims=True) * r / sr
    out_ref[pl.ds(S, S)] = jnp.reshape(thr, (S,))
    last = jnp.max(times)
    big = last + 1.0
    fb = jnp.where(last > bnd, last, bnd)
    li = lax.broadcasted_iota(jnp.int32, (1, 48), 1)
    out_ref[pl.ds(2 * S, 48)] = jnp.reshape(
        jnp.where(li < 16, big, jnp.where(li < 32, fb, 0.0)), (48,))


def _prelude(ifb, iast_t, exp_u, tle, bnd, ratio):
    return pl.pallas_call(
        _prelude_body,
        in_specs=[
            pl.BlockSpec(),
            pl.BlockSpec(),
            pl.BlockSpec(),
            pl.BlockSpec(memory_space=pltpu.SMEM),
            pl.BlockSpec(memory_space=pltpu.SMEM),
            pl.BlockSpec(memory_space=pltpu.SMEM),
        ],
        out_shape=jax.ShapeDtypeStruct((2 * S + 48,), jnp.float32),
    )(ifb, iast_t, exp_u, tle, bnd, ratio)


def _weights_body(w_ref):
    w_ref[...] = jnp.full((NUM_SAMPLE,), 1.0 / NUM_SAMPLE, jnp.float32)


def _weights():
    return pl.pallas_call(
        _weights_body,
        out_shape=jax.ShapeDtypeStruct((NUM_SAMPLE,), jnp.float32),
    )()


@functools.partial(
    pl.kernel,
    out_type=jax.ShapeDtypeStruct((NUM_SAMPLE,), jnp.float32),
    mesh=plsc.VectorSubcoreMesh(core_axis_name="c", subcore_axis_name="s"),
    scratch_types=[
        pltpu.VMEM((48,), jnp.float32),
        pltpu.VMEM((S,), jnp.float32),
        pltpu.VMEM((S,), jnp.float32),
        pltpu.VMEM((R, C0), jnp.float32),
        pltpu.VMEM((8, CC), jnp.float32),
        pltpu.VMEM((R,), jnp.float32),
        pltpu.SMEM((R,), jnp.int32),
        pltpu.SMEM((1,), jnp.int32),
        pltpu.SemaphoreType.DMA,
    ],
)
def _scan_kernel(prel_hbm, unif_hbm, rst_hbm,
                 sc_v, thr_v, times_v, buf_v, cbuf_v, rst_v, list_s, cnt_s,
                 sem):
    wid = lax.axis_index("s") * 2 + lax.axis_index("c")
    base = wid * R
    cp = pltpu.async_copy(unif_hbm.at[pl.ds(base, R), pl.ds(0, C0)], buf_v, sem)
    pltpu.sync_copy(prel_hbm.at[pl.ds(2 * S, 48)], sc_v)
    pltpu.sync_copy(prel_hbm.at[pl.ds(S, S)], thr_v)
    pltpu.sync_copy(prel_hbm.at[pl.ds(0, S)], times_v)
    bigv = sc_v[pl.ds(0, LANES)]
    fbv = sc_v[pl.ds(16, LANES)]
    lane = lax.broadcasted_iota(jnp.int32, (LANES,), 0)
    cnt_s[0] = 0
    cp.wait()

    def lane_min(x):
        for k in (1, 2, 4, 8):
            x = jnp.minimum(x, x.at[lane ^ k].get(mode="promise_in_bounds"))
        return x

    def fold(load_u, col0):
        acc = bigv
        for v in range(CC // LANES):
            u = load_u(v)
            t = thr_v[pl.ds(col0 + v * LANES, LANES)]
            tm = times_v[pl.ds(col0 + v * LANES, LANES)]
            acc = jnp.minimum(acc, jnp.where(u < t, tm, bigv))
        return acc

    def scalar_of(vec):
        return jnp.reshape(lax.slice(vec, (0,), (1,)), ())

    def append_if_unresolved(row, minv):
        c = cnt_s[0]
        list_s[c] = row
        cnt_s[0] = c + scalar_of(jnp.where(minv < bigv, 0, 1))

    def result_of(minv):
        return jnp.where(minv < bigv, minv, fbv)

    def row_fn(i, gvec):
        acc = fold(lambda v: buf_v[i, pl.ds(v * LANES, LANES)], 0)
        minv = lane_min(acc)
        append_if_unresolved(i, minv)
        gvec = jnp.where(lane == lax.rem(i, LANES), result_of(minv), gvec)

        @pl.when(lax.rem(i, LANES) == LANES - 1)
        def _():
            rst_v[pl.ds((i // LANES) * LANES, LANES)] = gvec

        return gvec

    lax.fori_loop(0, R, row_fn, jnp.zeros((LANES,), jnp.float32))

    def round_fn(r, carry):
        n = cnt_s[0]
        cnt_s[0] = 0
        col = C0 + r * CC

        def item_fn(k, carry2):
            row = list_s[k]
            r8 = pl.multiple_of(base + (row // 8) * 8, 8)
            pltpu.sync_copy(unif_hbm.at[pl.ds(r8, 8), pl.ds(col, CC)], cbuf_v)
            sub = lax.rem(row, 8)
            acc = fold(lambda v: cbuf_v[sub, pl.ds(v * LANES, LANES)], col)
            minv = lane_min(acc)
            append_if_unresolved(row, minv)
            g16 = (row // LANES) * LANES
            old = rst_v[pl.ds(g16, LANES)]
            sel = lane == lax.rem(row, LANES)
            rst_v[pl.ds(g16, LANES)] = jnp.where(sel, result_of(minv), old)
            return carry2

        lax.fori_loop(0, n, item_fn, 0)
        return carry

    lax.fori_loop(0, (S - C0) // CC, round_fn, 0)
    pltpu.sync_copy(rst_v, rst_hbm.at[pl.ds(base, R)])


def kernel(intensities_for_bound, intensities_at_sampled_times, exp_u,
           unif_numbers, time_last_event, boundary, ratio):
    iast_t = intensities_at_sampled_times.reshape(S, -1).T
    prel = _prelude(
        intensities_for_bound, iast_t, exp_u,
        time_last_event.reshape(1, 1), boundary.reshape(1, 1),
        ratio.reshape(1, 1))
    rst = _scan_kernel(prel, unif_numbers)
    return rst, _weights()
